# baseline (speedup 1.0000x reference)
import functools

import jax
import jax.numpy as jnp
from jax import lax
from jax.experimental import pallas as pl
from jax.experimental.pallas import tpu as pltpu
from jax.experimental.pallas import tpu_sc as plsc

B, T, LAG, NE = 1024, 200, 50, 32

OFF_M, OFF_D, OFF_H, OFF_DOW = 0, 13, 45, 69
OFF_POS, OFF_FUT, OFF_LOW = 76, 276, 327
N_ROWS = 330

NC, NS = 2, 16
NW = NC * NS
PAIRS = B * T
PAIRS_PER_W = PAIRS // NW
BLK = 128
N_BLK = PAIRS_PER_W // BLK
GROUPS = BLK // 16


def _emb_x_gather(comb, x_flat):
    mesh = plsc.VectorSubcoreMesh(core_axis_name="c", subcore_axis_name="s")

    @functools.partial(
        pl.kernel,
        out_type=jax.ShapeDtypeStruct((PAIRS * 8, NE), jnp.float32),
        mesh=mesh,
        compiler_params=pltpu.CompilerParams(
            use_tc_tiling_on_sc=False, needs_layout_passes=False
        ),
        scratch_types=[
            pltpu.VMEM((BLK * 6,), jnp.int32),
            pltpu.VMEM((GROUPS, 128), jnp.int32),
            pltpu.VMEM((BLK * 8, NE), jnp.float32),
            pltpu.SemaphoreType.DMA,
        ],
    )
    def k(comb_hbm, x_hbm, out_hbm, x_v, idx_v, rows_v, sem):
        wid = lax.axis_index("s") * NC + lax.axis_index("c")
        base_pair = wid * PAIRS_PER_W
        lane = lax.iota(jnp.int32, 16)

        def blk_body(i, carry):
            p0 = base_pair + i * BLK
            pltpu.sync_copy(x_hbm.at[pl.ds(p0 * 6, BLK * 6)], x_v)
            for g in range(GROUPS):
                loc = g * 16 + lane
                t = lax.rem(p0 + loc, T)
                gi = jnp.full((16,), g, jnp.int32)

                def put(c, vals, gi=gi):
                    plsc.store_scatter(idx_v, [gi, lane * 8 + c], vals)

                put(0, plsc.load_gather(x_v, [loc * 6 + 1]) + OFF_M)
                put(1, plsc.load_gather(x_v, [loc * 6 + 2]) + OFF_D)
                put(2, plsc.load_gather(x_v, [loc * 6 + 3]) + OFF_H)
                put(3, plsc.load_gather(x_v, [loc * 6 + 4]) + OFF_DOW)
                put(4, t + OFF_POS)
                put(5, jnp.maximum(t - (T - LAG - 1), 0) + OFF_FUT)
                put(6, jnp.where(t >= T - LAG, OFF_FUT + 1, OFF_FUT))
                put(7, plsc.load_gather(x_v, [loc * 6 + 5]) + OFF_LOW)
            copies = [
                pltpu.async_copy(
                    comb_hbm.at[idx_v.at[g]],
                    rows_v.at[pl.ds(g * 128, 128)],
                    sem,
                )
                for g in range(GROUPS)
            ]
            for cp in copies:
                cp.wait()
            pltpu.sync_copy(rows_v, out_hbm.at[pl.ds(p0 * 8, BLK * 8)])
            return carry

        lax.fori_loop(0, N_BLK, blk_body, 0)

    return k(comb, x_flat)


def _emb_y_proj(y, w_row, b_row):
    blk_b = 128
    grid = (B // blk_b,)

    def body(y_ref, w_ref, b_ref, past_ref, fut_ref):
        w = w_ref[0, :][None, None, :]
        bb = b_ref[0, :][None, None, :]
        past_ref[...] = y_ref[:, : T - LAG][:, :, None] * w + bb
        fut_ref[...] = y_ref[:, T - LAG - 1 : T - 1][:, :, None] * w + bb

    return pl.pallas_call(
        body,
        grid=grid,
        in_specs=[
            pl.BlockSpec((blk_b, T), lambda i: (i, 0)),
            pl.BlockSpec((1, NE), lambda i: (0, 0)),
            pl.BlockSpec((1, NE), lambda i: (0, 0)),
        ],
        out_specs=[
            pl.BlockSpec((blk_b, T - LAG, NE), lambda i: (i, 0, 0)),
            pl.BlockSpec((blk_b, LAG, NE), lambda i: (i, 0, 0)),
        ],
        out_shape=[
            jax.ShapeDtypeStruct((B, T - LAG, NE), jnp.float32),
            jax.ShapeDtypeStruct((B, LAG, NE), jnp.float32),
        ],
    )(y, w_row, b_row)


def kernel(x, y, emb_m, emb_d, emb_h, emb_dow, emb_pos, emb_future_pos, emb_low, w_y, b_y):
    comb = jnp.concatenate(
        [emb_m, emb_d, emb_h, emb_dow, emb_pos, emb_future_pos, emb_low], axis=0
    )
    rows = _emb_x_gather(comb, x.reshape(-1))
    emb_x = rows.reshape(B, T, 8, NE)
    emb_y_past, emb_y_fut = _emb_y_proj(
        y, w_y.reshape(1, NE), b_y.reshape(1, NE)
    )
    return (emb_x, emb_y_past, emb_y_fut)


# speedup vs baseline: 1.7559x; 1.7559x over previous
import functools

import jax
import jax.numpy as jnp
from jax import lax
from jax.experimental import pallas as pl
from jax.experimental.pallas import tpu as pltpu
from jax.experimental.pallas import tpu_sc as plsc

B, T, LAG, NE = 1024, 200, 50, 32

OFF_M, OFF_D, OFF_H, OFF_DOW = 0, 13, 45, 69
OFF_POS, OFF_FUT, OFF_LOW = 76, 276, 327
N_ROWS = 330

NC, NS = 2, 16
NW = NC * NS
PAIRS = B * T
PAIRS_PER_W = PAIRS // NW
BLK = 128
N_BLK = PAIRS_PER_W // BLK
GROUPS = BLK // 16
ROW_F32 = 8 * NE
BLK_F32 = BLK * ROW_F32


def _emb_x_gather(comb_flat, x_flat):
    mesh = plsc.VectorSubcoreMesh(core_axis_name="c", subcore_axis_name="s")

    @functools.partial(
        pl.kernel,
        out_type=jax.ShapeDtypeStruct((PAIRS * ROW_F32,), jnp.float32),
        mesh=mesh,
        compiler_params=pltpu.CompilerParams(
            use_tc_tiling_on_sc=False, needs_layout_passes=False
        ),
        scratch_types=[
            pltpu.VMEM((N_ROWS * NE,), jnp.float32),
            pltpu.VMEM((BLK * 6,), jnp.int32),
            pltpu.VMEM((BLK_F32,), jnp.float32),
            pltpu.VMEM((BLK_F32,), jnp.float32),
            pltpu.SemaphoreType.DMA,
            pltpu.SemaphoreType.DMA,
        ],
    )
    def k(comb_hbm, x_hbm, out_hbm, comb_v, x_v, rows0_v, rows1_v, sem0, sem1):
        rows_v = (rows0_v, rows1_v)
        sems = (sem0, sem1)
        wid = lax.axis_index("s") * NC + lax.axis_index("c")
        base_pair = wid * PAIRS_PER_W
        lane = lax.iota(jnp.int32, 16)

        pltpu.sync_copy(comb_hbm, comb_v)

        def fill_block(i, buf):
            p0 = base_pair + i * BLK
            pltpu.sync_copy(x_hbm.at[pl.ds(p0 * 6, BLK * 6)], x_v)

            def group_body(g, carry):
                loc = g * 16 + lane
                t = lax.rem(p0 + loc, T)

                def xg(col):
                    return plsc.load_gather(x_v, [loc * 6 + col])

                rb = (
                    (xg(1) + OFF_M) * NE,
                    (xg(2) + OFF_D) * NE,
                    (xg(3) + OFF_H) * NE,
                    (xg(4) + OFF_DOW) * NE,
                    (t + OFF_POS) * NE,
                    (jnp.maximum(t - (T - LAG - 1), 0) + OFF_FUT) * NE,
                    jnp.where(t >= T - LAG, OFF_FUT + 1, OFF_FUT) * NE,
                    (xg(5) + OFF_LOW) * NE,
                )
                dbase = g * (16 * ROW_F32) + lane * ROW_F32
                for c in range(8):
                    sb = dbase + c * NE
                    for e in range(NE):
                        v = plsc.load_gather(comb_v, [rb[c] + e])
                        plsc.store_scatter(buf, [sb + e], v)
                return carry

            lax.fori_loop(0, GROUPS, group_body, 0)

        def flush_block(i, b):
            p0 = base_pair + i * BLK
            return pltpu.async_copy(
                rows_v[b], out_hbm.at[pl.ds(p0 * ROW_F32, BLK_F32)], sems[b]
            )

        def drain(b):
            pltpu.make_async_copy(
                rows_v[b], out_hbm.at[pl.ds(0, BLK_F32)], sems[b]
            ).wait()

        fill_block(0, rows_v[0])
        flush_block(0, 0)
        fill_block(1, rows_v[1])
        flush_block(1, 1)

        def steady(j, carry):
            for b in range(2):
                i = 2 * j + 2 + b
                drain(b)
                fill_block(i, rows_v[b])
                flush_block(i, b)
            return carry

        lax.fori_loop(0, (N_BLK - 2) // 2, steady, 0)
        drain(0)
        drain(1)

    return k(comb_flat, x_flat)


def _emb_y_proj(y, w_row, b_row):
    blk_b = 128
    grid = (B // blk_b,)

    def body(y_ref, w_ref, b_ref, past_ref, fut_ref):
        w = w_ref[0, :][None, None, :]
        bb = b_ref[0, :][None, None, :]
        past_ref[...] = y_ref[:, : T - LAG][:, :, None] * w + bb
        fut_ref[...] = y_ref[:, T - LAG - 1 : T - 1][:, :, None] * w + bb

    return pl.pallas_call(
        body,
        grid=grid,
        in_specs=[
            pl.BlockSpec((blk_b, T), lambda i: (i, 0)),
            pl.BlockSpec((1, NE), lambda i: (0, 0)),
            pl.BlockSpec((1, NE), lambda i: (0, 0)),
        ],
        out_specs=[
            pl.BlockSpec((blk_b, T - LAG, NE), lambda i: (i, 0, 0)),
            pl.BlockSpec((blk_b, LAG, NE), lambda i: (i, 0, 0)),
        ],
        out_shape=[
            jax.ShapeDtypeStruct((B, T - LAG, NE), jnp.float32),
            jax.ShapeDtypeStruct((B, LAG, NE), jnp.float32),
        ],
    )(y, w_row, b_row)


def kernel(x, y, emb_m, emb_d, emb_h, emb_dow, emb_pos, emb_future_pos, emb_low, w_y, b_y):
    comb = jnp.concatenate(
        [emb_m, emb_d, emb_h, emb_dow, emb_pos, emb_future_pos, emb_low], axis=0
    )
    flat = _emb_x_gather(comb.reshape(-1), x.reshape(-1))
    emb_x = flat.reshape(B, T, 8, NE)
    emb_y_past, emb_y_fut = _emb_y_proj(
        y, w_y.reshape(1, NE), b_y.reshape(1, NE)
    )
    return (emb_x, emb_y_past, emb_y_fut)


# speedup vs baseline: 4.3054x; 2.4520x over previous
import functools

import jax
import jax.numpy as jnp
from jax import lax
from jax.experimental import pallas as pl
from jax.experimental.pallas import tpu as pltpu
from jax.experimental.pallas import tpu_sc as plsc

B, T, LAG, NE = 1024, 200, 50, 32

OFF_M, OFF_D, OFF_H, OFF_DOW = 0, 13, 45, 69
OFF_POS, OFF_FUT, OFF_LOW = 76, 276, 327
N_ROWS = 330

NC, NS = 2, 16
NW = NC * NS
PAIRS = B * T
PAIRS_PER_W = PAIRS // NW
BLK = 128
N_BLK = PAIRS_PER_W // BLK
GROUPS = BLK // 16
ROW_F32 = 8 * NE
BLK_F32 = BLK * ROW_F32


def _emb_x_gather(comb_flat, x_flat):
    mesh = plsc.VectorSubcoreMesh(core_axis_name="c", subcore_axis_name="s")

    @functools.partial(
        pl.kernel,
        out_type=jax.ShapeDtypeStruct((PAIRS * ROW_F32,), jnp.float32),
        mesh=mesh,
        compiler_params=pltpu.CompilerParams(
            use_tc_tiling_on_sc=False, needs_layout_passes=False
        ),
        scratch_types=[
            pltpu.VMEM((N_ROWS * NE,), jnp.float32),
            pltpu.VMEM((BLK * 6 + 16,), jnp.int32),
            pltpu.VMEM((BLK_F32,), jnp.float32),
            pltpu.VMEM((BLK_F32,), jnp.float32),
            pltpu.SemaphoreType.DMA,
            pltpu.SemaphoreType.DMA,
        ],
    )
    def k(comb_hbm, x_hbm, out_hbm, comb_v, x_v, rows0_v, rows1_v, sem0, sem1):
        rows_v = (rows0_v, rows1_v)
        sems = (sem0, sem1)
        wid = lax.axis_index("s") * NC + lax.axis_index("c")
        base_pair = wid * PAIRS_PER_W

        pltpu.sync_copy(comb_hbm, comb_v)

        def fill_block(i, buf):
            p0 = base_pair + i * BLK
            pltpu.sync_copy(
                x_hbm.at[pl.ds(p0 * 6, BLK * 6)], x_v.at[pl.ds(0, BLK * 6)]
            )

            @plsc.parallel_loop(0, BLK, unroll=4)
            def pair_body(p):
                t = lax.rem(p0 + p, T)
                xv = x_v[pl.ds(p * 6, 16)]
                rows = (
                    xv[1] + OFF_M,
                    xv[2] + OFF_D,
                    xv[3] + OFF_H,
                    xv[4] + OFF_DOW,
                    t + OFF_POS,
                    jnp.maximum(t - (T - LAG - 1), 0) + OFF_FUT,
                    jnp.where(t >= T - LAG, OFF_FUT + 1, OFF_FUT),
                    xv[5] + OFF_LOW,
                )
                base = p * ROW_F32
                for c in range(8):
                    src = rows[c] * NE
                    for h in range(0, NE, 16):
                        buf[pl.ds(base + c * NE + h, 16)] = comb_v[
                            pl.ds(src + h, 16)
                        ]

        def flush_block(i, b):
            p0 = base_pair + i * BLK
            return pltpu.async_copy(
                rows_v[b], out_hbm.at[pl.ds(p0 * ROW_F32, BLK_F32)], sems[b]
            )

        def drain(b):
            pltpu.make_async_copy(
                rows_v[b], out_hbm.at[pl.ds(0, BLK_F32)], sems[b]
            ).wait()

        fill_block(0, rows_v[0])
        flush_block(0, 0)
        fill_block(1, rows_v[1])
        flush_block(1, 1)

        def steady(j, carry):
            for b in range(2):
                i = 2 * j + 2 + b
                drain(b)
                fill_block(i, rows_v[b])
                flush_block(i, b)
            return carry

        lax.fori_loop(0, (N_BLK - 2) // 2, steady, 0)
        drain(0)
        drain(1)

    return k(comb_flat, x_flat)


def _emb_y_proj(y, w_row, b_row):
    blk_b = 128
    grid = (B // blk_b,)

    def body(y_ref, w_ref, b_ref, past_ref, fut_ref):
        w = w_ref[0, :][None, None, :]
        bb = b_ref[0, :][None, None, :]
        past_ref[...] = y_ref[:, : T - LAG][:, :, None] * w + bb
        fut_ref[...] = y_ref[:, T - LAG - 1 : T - 1][:, :, None] * w + bb

    return pl.pallas_call(
        body,
        grid=grid,
        in_specs=[
            pl.BlockSpec((blk_b, T), lambda i: (i, 0)),
            pl.BlockSpec((1, NE), lambda i: (0, 0)),
            pl.BlockSpec((1, NE), lambda i: (0, 0)),
        ],
        out_specs=[
            pl.BlockSpec((blk_b, T - LAG, NE), lambda i: (i, 0, 0)),
            pl.BlockSpec((blk_b, LAG, NE), lambda i: (i, 0, 0)),
        ],
        out_shape=[
            jax.ShapeDtypeStruct((B, T - LAG, NE), jnp.float32),
            jax.ShapeDtypeStruct((B, LAG, NE), jnp.float32),
        ],
    )(y, w_row, b_row)


def kernel(x, y, emb_m, emb_d, emb_h, emb_dow, emb_pos, emb_future_pos, emb_low, w_y, b_y):
    comb = jnp.concatenate(
        [emb_m, emb_d, emb_h, emb_dow, emb_pos, emb_future_pos, emb_low], axis=0
    )
    flat = _emb_x_gather(comb.reshape(-1), x.reshape(-1))
    emb_x = flat.reshape(B, T, 8, NE)
    emb_y_past, emb_y_fut = _emb_y_proj(
        y, w_y.reshape(1, NE), b_y.reshape(1, NE)
    )
    return (emb_x, emb_y_past, emb_y_fut)


# speedup vs baseline: 9.0205x; 2.0951x over previous
import functools

import jax
import jax.numpy as jnp
from jax import lax
from jax.experimental import pallas as pl
from jax.experimental.pallas import tpu as pltpu
from jax.experimental.pallas import tpu_sc as plsc

B, T, LAG, NE = 1024, 200, 50, 32

OFF_M, OFF_D, OFF_H, OFF_DOW = 0, 13, 45, 69
OFF_POS, OFF_FUT, OFF_LOW = 76, 276, 327
N_ROWS = 330

NC, NS = 2, 16
NW = NC * NS
PLANES = T * 8
PL_PER_W = PLANES // NW


def _emb_x_gather(comb_flat, xt_flat):
    mesh = plsc.VectorSubcoreMesh(core_axis_name="c", subcore_axis_name="s")

    @functools.partial(
        pl.kernel,
        out_type=jax.ShapeDtypeStruct((T, 8, NE, B), jnp.float32),
        mesh=mesh,
        compiler_params=pltpu.CompilerParams(
            use_tc_tiling_on_sc=True, needs_layout_passes=False
        ),
        scratch_types=[
            pltpu.VMEM((N_ROWS * NE,), jnp.float32),
            pltpu.VMEM((B,), jnp.int32),
            pltpu.VMEM((NE, B), jnp.float32),
            pltpu.VMEM((NE, B), jnp.float32),
            pltpu.SemaphoreType.DMA,
            pltpu.SemaphoreType.DMA,
        ],
    )
    def k(comb_hbm, xt_hbm, out_hbm, comb_v, xrow_v, pl0_v, pl1_v, sem0, sem1):
        plane_v = (pl0_v, pl1_v)
        sems = (sem0, sem1)
        wid = lax.axis_index("s") * NC + lax.axis_index("c")
        base_plane = wid * PL_PER_W

        pltpu.sync_copy(comb_hbm, comb_v)

        def plane_tc(i):
            pi = base_plane + i
            return pi // 8, lax.rem(pi, 8)

        def fill_plane(i, buf):
            t, c = plane_tc(i)
            col = jnp.where(c < 4, c + 1, jnp.where(c == 7, 5, 0))
            offc = jnp.where(
                c == 0,
                OFF_M,
                jnp.where(
                    c == 1,
                    OFF_D,
                    jnp.where(c == 2, OFF_H, jnp.where(c == 3, OFF_DOW, OFF_LOW)),
                ),
            )
            rowt = jnp.where(
                c == 4,
                OFF_POS + t,
                jnp.where(
                    c == 5,
                    OFF_FUT + jnp.maximum(t - (T - LAG - 1), 0),
                    OFF_FUT + (t >= T - LAG).astype(jnp.int32),
                ),
            )
            is_bcast = (c >= 4) & (c <= 6)
            pltpu.sync_copy(xt_hbm.at[pl.ds((t * 6 + col) * B, B)], xrow_v)

            @plsc.parallel_loop(0, B // 16, unroll=2)
            def grp(g):
                xv = xrow_v[pl.ds(g * 16, 16)]
                row16 = jnp.where(is_bcast, rowt, xv + offc)
                rbase = row16 * NE
                for e in range(NE):
                    buf[e, pl.ds(g * 16, 16)] = plsc.load_gather(
                        comb_v, [rbase + e]
                    )

        def flush_plane(i, b):
            t, c = plane_tc(i)
            return pltpu.async_copy(plane_v[b], out_hbm.at[t, c], sems[b])

        def drain(b):
            pltpu.make_async_copy(plane_v[b], out_hbm.at[0, 0], sems[b]).wait()

        fill_plane(0, plane_v[0])
        flush_plane(0, 0)
        fill_plane(1, plane_v[1])
        flush_plane(1, 1)

        def steady(j, carry):
            for b in range(2):
                i = 2 * j + 2 + b
                drain(b)
                fill_plane(i, plane_v[b])
                flush_plane(i, b)
            return carry

        lax.fori_loop(0, (PL_PER_W - 2) // 2, steady, 0)
        drain(0)
        drain(1)

    return k(comb_flat, xt_flat)


def _emb_y_proj(yt, w_row, b_row):
    blk_b = 256
    grid = (B // blk_b,)

    def body(y_ref, w_ref, b_ref, past_ref, fut_ref):
        w = w_ref[0, :][None, :, None]
        bb = b_ref[0, :][None, :, None]
        past_ref[...] = y_ref[: T - LAG][:, None, :] * w + bb
        fut_ref[...] = y_ref[T - LAG - 1 : T - 1][:, None, :] * w + bb

    return pl.pallas_call(
        body,
        grid=grid,
        in_specs=[
            pl.BlockSpec((T, blk_b), lambda i: (0, i)),
            pl.BlockSpec((1, NE), lambda i: (0, 0)),
            pl.BlockSpec((1, NE), lambda i: (0, 0)),
        ],
        out_specs=[
            pl.BlockSpec((T - LAG, NE, blk_b), lambda i: (0, 0, i)),
            pl.BlockSpec((LAG, NE, blk_b), lambda i: (0, 0, i)),
        ],
        out_shape=[
            jax.ShapeDtypeStruct((T - LAG, NE, B), jnp.float32),
            jax.ShapeDtypeStruct((LAG, NE, B), jnp.float32),
        ],
    )(yt, w_row, b_row)


def kernel(x, y, emb_m, emb_d, emb_h, emb_dow, emb_pos, emb_future_pos, emb_low, w_y, b_y):
    comb = jnp.concatenate(
        [emb_m, emb_d, emb_h, emb_dow, emb_pos, emb_future_pos, emb_low], axis=0
    )
    xt = x.transpose(1, 2, 0)
    otc = _emb_x_gather(comb.reshape(-1), xt.reshape(-1))
    emb_x = otc.transpose(3, 0, 1, 2)
    past_t, fut_t = _emb_y_proj(y.T, w_y.reshape(1, NE), b_y.reshape(1, NE))
    return (emb_x, past_t.transpose(2, 0, 1), fut_t.transpose(2, 0, 1))


# speedup vs baseline: 9.6265x; 1.0672x over previous
import functools

import jax
import jax.numpy as jnp
from jax import lax
from jax.experimental import pallas as pl
from jax.experimental.pallas import tpu as pltpu
from jax.experimental.pallas import tpu_sc as plsc

B, T, LAG, NE = 1024, 200, 50, 32

OFF_M, OFF_D, OFF_H, OFF_DOW = 0, 13, 45, 69
OFF_POS, OFF_FUT, OFF_LOW = 76, 276, 327
N_ROWS = 330

NC, NS = 2, 16
NW = NC * NS
PLANES = T * 8
PL_PER_W = PLANES // NW


def _emb_x_gather(comb_flat, xt_flat):
    mesh = plsc.VectorSubcoreMesh(core_axis_name="c", subcore_axis_name="s")

    @functools.partial(
        pl.kernel,
        out_type=jax.ShapeDtypeStruct((T, 8, NE, B), jnp.float32),
        mesh=mesh,
        compiler_params=pltpu.CompilerParams(
            use_tc_tiling_on_sc=True, needs_layout_passes=False
        ),
        scratch_types=[
            pltpu.VMEM((N_ROWS * NE,), jnp.float32),
            pltpu.VMEM((B,), jnp.int32),
            pltpu.VMEM((B,), jnp.int32),
            pltpu.VMEM((NE, B), jnp.float32),
            pltpu.VMEM((NE, B), jnp.float32),
            pltpu.SemaphoreType.DMA,
            pltpu.SemaphoreType.DMA,
            pltpu.SemaphoreType.DMA,
            pltpu.SemaphoreType.DMA,
        ],
    )
    def k(comb_hbm, xt_hbm, out_hbm, comb_v, x0_v, x1_v, pl0_v, pl1_v,
          sem0, sem1, semx0, semx1):
        plane_v = (pl0_v, pl1_v)
        xrow_v = (x0_v, x1_v)
        sems = (sem0, sem1)
        semx = (semx0, semx1)
        wid = lax.axis_index("s") * NC + lax.axis_index("c")
        base_plane = wid * PL_PER_W

        pltpu.sync_copy(comb_hbm, comb_v)

        def plane_tc(i):
            pi = base_plane + i
            return pi // 8, lax.rem(pi, 8)

        def prefetch_x(i, xb):
            pi = jnp.minimum(base_plane + i, PLANES - 1)
            t = pi // 8
            c = lax.rem(pi, 8)
            col = jnp.where(c < 4, c + 1, jnp.where(c == 7, 5, 0))
            pltpu.async_copy(
                xt_hbm.at[pl.ds((t * 6 + col) * B, B)], xrow_v[xb], semx[xb]
            )

        def wait_x(xb):
            pltpu.make_async_copy(
                xt_hbm.at[pl.ds(0, B)], xrow_v[xb], semx[xb]
            ).wait()

        def fill_plane(i, b):
            t, c = plane_tc(i)
            buf = plane_v[b]
            offc = jnp.where(
                c == 0,
                OFF_M,
                jnp.where(
                    c == 1,
                    OFF_D,
                    jnp.where(c == 2, OFF_H, jnp.where(c == 3, OFF_DOW, OFF_LOW)),
                ),
            )
            rowt = jnp.where(
                c == 4,
                OFF_POS + t,
                jnp.where(
                    c == 5,
                    OFF_FUT + jnp.maximum(t - (T - LAG - 1), 0),
                    OFF_FUT + (t >= T - LAG).astype(jnp.int32),
                ),
            )
            is_bcast = (c >= 4) & (c <= 6)

            def data_body(_):
                @plsc.parallel_loop(0, B // 16, unroll=2)
                def grp(g):
                    xv = xrow_v[b][pl.ds(g * 16, 16)]
                    rbase = (xv + offc) * NE
                    for e in range(NE):
                        buf[e, pl.ds(g * 16, 16)] = plsc.load_gather(
                            comb_v, [rbase + e]
                        )
                return 0

            def bcast_body(_):
                rbase = jnp.full((16,), rowt * NE, jnp.int32)

                def e_body(e, carry):
                    splat = plsc.load_gather(comb_v, [rbase + e])
                    for g in range(B // 16):
                        buf[e, pl.ds(g * 16, 16)] = splat
                    return carry

                return lax.fori_loop(0, NE, e_body, 0)

            lax.cond(is_bcast, bcast_body, data_body, 0)

        def flush_plane(i, b):
            t, c = plane_tc(i)
            return pltpu.async_copy(plane_v[b], out_hbm.at[t, c], sems[b])

        def drain(b):
            pltpu.make_async_copy(plane_v[b], out_hbm.at[0, 0], sems[b]).wait()

        prefetch_x(0, 0)
        prefetch_x(1, 1)
        for i in range(2):
            wait_x(i)
            fill_plane(i, i)
            flush_plane(i, i)
            prefetch_x(i + 2, i)

        def steady(j, carry):
            for b in range(2):
                i = 2 * j + 2 + b
                drain(b)
                wait_x(b)
                fill_plane(i, b)
                flush_plane(i, b)
                prefetch_x(i + 2, b)
            return carry

        lax.fori_loop(0, (PL_PER_W - 2) // 2, steady, 0)
        wait_x(0)
        wait_x(1)
        drain(0)
        drain(1)

    return k(comb_flat, xt_flat)


def _emb_y_proj(yt, w_row, b_row):
    blk_b = 256
    grid = (B // blk_b,)

    def body(y_ref, w_ref, b_ref, past_ref, fut_ref):
        w = w_ref[0, :][None, :, None]
        bb = b_ref[0, :][None, :, None]
        past_ref[...] = y_ref[: T - LAG][:, None, :] * w + bb
        fut_ref[...] = y_ref[T - LAG - 1 : T - 1][:, None, :] * w + bb

    return pl.pallas_call(
        body,
        grid=grid,
        in_specs=[
            pl.BlockSpec((T, blk_b), lambda i: (0, i)),
            pl.BlockSpec((1, NE), lambda i: (0, 0)),
            pl.BlockSpec((1, NE), lambda i: (0, 0)),
        ],
        out_specs=[
            pl.BlockSpec((T - LAG, NE, blk_b), lambda i: (0, 0, i)),
            pl.BlockSpec((LAG, NE, blk_b), lambda i: (0, 0, i)),
        ],
        out_shape=[
            jax.ShapeDtypeStruct((T - LAG, NE, B), jnp.float32),
            jax.ShapeDtypeStruct((LAG, NE, B), jnp.float32),
        ],
    )(yt, w_row, b_row)


def kernel(x, y, emb_m, emb_d, emb_h, emb_dow, emb_pos, emb_future_pos, emb_low, w_y, b_y):
    comb = jnp.concatenate(
        [emb_m, emb_d, emb_h, emb_dow, emb_pos, emb_future_pos, emb_low], axis=0
    )
    xt = x.transpose(1, 2, 0)
    otc = _emb_x_gather(comb.reshape(-1), xt.reshape(-1))
    emb_x = otc.transpose(3, 0, 1, 2)
    past_t, fut_t = _emb_y_proj(y.T, w_y.reshape(1, NE), b_y.reshape(1, NE))
    return (emb_x, past_t.transpose(2, 0, 1), fut_t.transpose(2, 0, 1))


# speedup vs baseline: 37.8892x; 3.9359x over previous
import functools

import jax
import jax.numpy as jnp
from jax import lax
from jax.experimental import pallas as pl
from jax.experimental.pallas import tpu as pltpu
from jax.experimental.pallas import tpu_sc as plsc

B, T, LAG, NE = 1024, 200, 50, 32

OFF_M, OFF_D, OFF_H, OFF_DOW = 0, 13, 45, 69
OFF_POS, OFF_FUT, OFF_LOW = 76, 276, 327
N_ROWS = 330

NC, NS = 2, 16
NW = NC * NS
PLANES = T * 8
PL_PER_W = PLANES // NW


N_ROWS_PAD = 336


def _emb_x_gather(combt_flat, xt_flat):
    mesh = plsc.VectorSubcoreMesh(core_axis_name="c", subcore_axis_name="s")

    @functools.partial(
        pl.kernel,
        out_type=jax.ShapeDtypeStruct((T, 8, NE, B), jnp.float32),
        mesh=mesh,
        compiler_params=pltpu.CompilerParams(
            use_tc_tiling_on_sc=True, needs_layout_passes=False
        ),
        scratch_types=[
            pltpu.VMEM((NE * N_ROWS_PAD,), jnp.float32),
            pltpu.VMEM((B,), jnp.int32),
            pltpu.VMEM((B,), jnp.int32),
            pltpu.VMEM((NE, B), jnp.float32),
            pltpu.VMEM((NE, B), jnp.float32),
            pltpu.SemaphoreType.DMA,
            pltpu.SemaphoreType.DMA,
            pltpu.SemaphoreType.DMA,
            pltpu.SemaphoreType.DMA,
        ],
    )
    def k(comb_hbm, xt_hbm, out_hbm, comb_v, x0_v, x1_v, pl0_v, pl1_v,
          sem0, sem1, semx0, semx1):
        plane_v = (pl0_v, pl1_v)
        xrow_v = (x0_v, x1_v)
        sems = (sem0, sem1)
        semx = (semx0, semx1)
        wid = lax.axis_index("s") * NC + lax.axis_index("c")
        base_plane = wid * PL_PER_W

        pltpu.sync_copy(comb_hbm, comb_v)

        def plane_tc(i):
            pi = base_plane + i
            return pi // 8, lax.rem(pi, 8)

        def prefetch_x(i, xb):
            pi = jnp.minimum(base_plane + i, PLANES - 1)
            t = pi // 8
            c = lax.rem(pi, 8)
            col = jnp.where(c < 4, c + 1, jnp.where(c == 7, 5, 0))
            pltpu.async_copy(
                xt_hbm.at[pl.ds((t * 6 + col) * B, B)], xrow_v[xb], semx[xb]
            )

        def wait_x(xb):
            pltpu.make_async_copy(
                xt_hbm.at[pl.ds(0, B)], xrow_v[xb], semx[xb]
            ).wait()

        def fill_plane(i, b):
            t, c = plane_tc(i)
            buf = plane_v[b]
            offc = jnp.where(
                c == 0,
                OFF_M,
                jnp.where(
                    c == 1,
                    OFF_D,
                    jnp.where(c == 2, OFF_H, jnp.where(c == 3, OFF_DOW, OFF_LOW)),
                ),
            )
            rowt = jnp.where(
                c == 4,
                OFF_POS + t,
                jnp.where(
                    c == 5,
                    OFF_FUT + jnp.maximum(t - (T - LAG - 1), 0),
                    OFF_FUT + (t >= T - LAG).astype(jnp.int32),
                ),
            )
            is_bcast = (c >= 4) & (c <= 6)

            def data_body(_):
                @plsc.parallel_loop(0, B // 16, unroll=4)
                def grp(g):
                    xv = xrow_v[b][pl.ds(g * 16, 16)]
                    row16 = xv + offc
                    for e in range(NE):
                        buf[e, pl.ds(g * 16, 16)] = plsc.load_gather(
                            comb_v.at[pl.ds(e * N_ROWS_PAD, N_ROWS_PAD)],
                            [row16],
                        )
                return 0

            def bcast_body(_):
                rbase = jnp.full((16,), rowt, jnp.int32)

                def e_body(e, carry):
                    off = pl.multiple_of(e * N_ROWS_PAD, 8)
                    splat = plsc.load_gather(
                        comb_v.at[pl.ds(off, N_ROWS_PAD)], [rbase]
                    )
                    for g in range(B // 16):
                        buf[e, pl.ds(g * 16, 16)] = splat
                    return carry

                return lax.fori_loop(0, NE, e_body, 0)

            lax.cond(is_bcast, bcast_body, data_body, 0)

        def flush_plane(i, b):
            t, c = plane_tc(i)
            return pltpu.async_copy(plane_v[b], out_hbm.at[t, c], sems[b])

        def drain(b):
            pltpu.make_async_copy(plane_v[b], out_hbm.at[0, 0], sems[b]).wait()

        prefetch_x(0, 0)
        prefetch_x(1, 1)
        for i in range(2):
            wait_x(i)
            fill_plane(i, i)
            flush_plane(i, i)
            prefetch_x(i + 2, i)

        def steady(j, carry):
            for b in range(2):
                i = 2 * j + 2 + b
                drain(b)
                wait_x(b)
                fill_plane(i, b)
                flush_plane(i, b)
                prefetch_x(i + 2, b)
            return carry

        lax.fori_loop(0, (PL_PER_W - 2) // 2, steady, 0)
        wait_x(0)
        wait_x(1)
        drain(0)
        drain(1)

    return k(combt_flat, xt_flat)


def _emb_y_proj(yt, w_row, b_row):
    blk_b = 256
    grid = (B // blk_b,)

    def body(y_ref, w_ref, b_ref, past_ref, fut_ref):
        w = w_ref[0, :][None, :, None]
        bb = b_ref[0, :][None, :, None]
        past_ref[...] = y_ref[: T - LAG][:, None, :] * w + bb
        fut_ref[...] = y_ref[T - LAG - 1 : T - 1][:, None, :] * w + bb

    return pl.pallas_call(
        body,
        grid=grid,
        in_specs=[
            pl.BlockSpec((T, blk_b), lambda i: (0, i)),
            pl.BlockSpec((1, NE), lambda i: (0, 0)),
            pl.BlockSpec((1, NE), lambda i: (0, 0)),
        ],
        out_specs=[
            pl.BlockSpec((T - LAG, NE, blk_b), lambda i: (0, 0, i)),
            pl.BlockSpec((LAG, NE, blk_b), lambda i: (0, 0, i)),
        ],
        out_shape=[
            jax.ShapeDtypeStruct((T - LAG, NE, B), jnp.float32),
            jax.ShapeDtypeStruct((LAG, NE, B), jnp.float32),
        ],
    )(yt, w_row, b_row)


def kernel(x, y, emb_m, emb_d, emb_h, emb_dow, emb_pos, emb_future_pos, emb_low, w_y, b_y):
    comb = jnp.concatenate(
        [emb_m, emb_d, emb_h, emb_dow, emb_pos, emb_future_pos, emb_low], axis=0
    )
    combt = jnp.pad(comb.T, ((0, 0), (0, N_ROWS_PAD - N_ROWS)))
    xt = x.transpose(1, 2, 0)
    otc = _emb_x_gather(combt.reshape(-1), xt.reshape(-1))
    emb_x = otc.transpose(3, 0, 1, 2)
    past_t, fut_t = _emb_y_proj(y.T, w_y.reshape(1, NE), b_y.reshape(1, NE))
    return (emb_x, past_t.transpose(2, 0, 1), fut_t.transpose(2, 0, 1))
